# Initial kernel scaffold; baseline (speedup 1.0000x reference)
import functools

import jax
import jax.numpy as jnp
from jax import lax
from jax.experimental import pallas as pl
from jax.experimental.pallas import tpu as pltpu
from jax.experimental.pallas import tpu_sc as plsc

N = 10000
E = 320000
D = 128
DE = 16

NP_ = 10240
CH = 128
NTILES = 32
CPT = 80
EPT = CPT * CH
EP = NTILES * EPT
RPS = NP_ // 16

_mesh = plsc.VectorSubcoreMesh(
    core_axis_name="c", subcore_axis_name="s", num_cores=2, num_subcores=16
)


def _sc_body(h_hbm, p_hbm, src_hbm, dst_hbm, out_hbm,
             acc, src_v, dst_v, g_v, p_v, sem):
    c = lax.axis_index("c")
    s = lax.axis_index("s")
    wid = s * 2 + c

    zeros16 = jnp.zeros((16,), jnp.float32)

    @pl.loop(0, CH)
    def _(r):
        for cc in range(8):
            g_v[r, pl.ds(cc * 16, 16)] = zeros16

    for t in range(RPS // CH):
        pltpu.sync_copy(g_v, acc.at[pl.ds(s * RPS + t * CH, CH)])
    plsc.subcore_barrier()

    pltpu.sync_copy(src_hbm.at[wid], src_v)
    pltpu.sync_copy(dst_hbm.at[wid], dst_v)
    ebase = wid * EPT

    @pl.loop(0, CPT)
    def _(j):
        pltpu.async_copy(h_hbm.at[src_v.at[j]], g_v, sem).wait()
        pltpu.sync_copy(p_hbm.at[pl.ds(ebase + j * CH, CH)], p_v)

        @pl.loop(0, CH)
        def _(r):
            for cc in range(8):
                sl = pl.ds(cc * 16, 16)
                g_v[r, sl] = jnp.maximum(g_v[r, sl] + p_v[r, sl], 0.0)

        pltpu.sync_copy(g_v, acc.at[dst_v.at[j]], add=True)

    plsc.subcore_barrier()
    pltpu.sync_copy(acc.at[pl.ds(s * RPS, RPS)],
                    out_hbm.at[pl.ds(c * NP_ + s * RPS, RPS)])


_sc_aggregate = functools.partial(
    pl.kernel,
    out_type=jax.ShapeDtypeStruct((2 * NP_, D), jnp.float32),
    mesh=_mesh,
    scratch_types=[
        pltpu.VMEM_SHARED((NP_, D), jnp.float32),
        pltpu.VMEM((CPT, CH), jnp.int32),
        pltpu.VMEM((CPT, CH), jnp.int32),
        pltpu.VMEM((CH, D), jnp.float32),
        pltpu.VMEM((CH, D), jnp.float32),
        pltpu.SemaphoreType.DMA,
    ],
)(_sc_body)


def _mm_body(x_ref, w_ref, o_ref):
    o_ref[...] = jax.lax.dot_general(
        x_ref[...], w_ref[...], (((1,), (0,)), ((), ())),
        preferred_element_type=jnp.float32)


def _mm_bias_body(x_ref, w_ref, b_ref, o_ref):
    o_ref[...] = jax.lax.dot_general(
        x_ref[...], w_ref[...], (((1,), (0,)), ((), ())),
        preferred_element_type=jnp.float32) + b_ref[...]


def _final_body(x_ref, a0_ref, a1_ref, wa_ref, wb_ref, b_ref, o_ref):
    agg = a0_ref[...] + a1_ref[...]
    acc = jax.lax.dot_general(
        x_ref[...], wa_ref[...], (((1,), (0,)), ((), ())),
        preferred_element_type=jnp.float32)
    acc += jax.lax.dot_general(
        agg, wb_ref[...], (((1,), (0,)), ((), ())),
        preferred_element_type=jnp.float32)
    o_ref[...] = jnp.maximum(acc + b_ref[...], 0.0)


def _tc_matmul(x, w, block_rows):
    m = x.shape[0]
    return pl.pallas_call(
        _mm_body,
        grid=(m // block_rows,),
        in_specs=[
            pl.BlockSpec((block_rows, x.shape[1]), lambda i: (i, 0)),
            pl.BlockSpec((w.shape[0], w.shape[1]), lambda i: (0, 0)),
        ],
        out_specs=pl.BlockSpec((block_rows, w.shape[1]), lambda i: (i, 0)),
        out_shape=jax.ShapeDtypeStruct((m, w.shape[1]), jnp.float32),
    )(x, w)


def _tc_matmul_bias(x, w, b, block_rows):
    m = x.shape[0]
    return pl.pallas_call(
        _mm_bias_body,
        grid=(m // block_rows,),
        in_specs=[
            pl.BlockSpec((block_rows, x.shape[1]), lambda i: (i, 0)),
            pl.BlockSpec((w.shape[0], w.shape[1]), lambda i: (0, 0)),
            pl.BlockSpec((1, w.shape[1]), lambda i: (0, 0)),
        ],
        out_specs=pl.BlockSpec((block_rows, w.shape[1]), lambda i: (i, 0)),
        out_shape=jax.ShapeDtypeStruct((m, w.shape[1]), jnp.float32),
    )(x, w, b)


def _tc_final(x_ap, partials, wa, wb, b, block_rows):
    nblk = NP_ // block_rows
    return pl.pallas_call(
        _final_body,
        grid=(nblk,),
        in_specs=[
            pl.BlockSpec((block_rows, D), lambda i: (i, 0)),
            pl.BlockSpec((block_rows, D), lambda i: (i, 0)),
            pl.BlockSpec((block_rows, D), lambda i: (i + NP_ // block_rows, 0)),
            pl.BlockSpec((D, D), lambda i: (0, 0)),
            pl.BlockSpec((D, D), lambda i: (0, 0)),
            pl.BlockSpec((1, D), lambda i: (0, 0)),
        ],
        out_specs=pl.BlockSpec((block_rows, D), lambda i: (i, 0)),
        out_shape=jax.ShapeDtypeStruct((NP_, D), jnp.float32),
    )(x_ap, partials, partials, wa, wb, b)


def kernel(x_ue, x_ap, edge_index, edge_attr, W1, b1, W2, b2):
    src = edge_index[0].astype(jnp.int32)
    dst = edge_index[1].astype(jnp.int32)

    pad_e = EP - E
    src_p = jnp.concatenate([src, jnp.zeros((pad_e,), jnp.int32)])
    dst_p = jnp.concatenate([dst, jnp.full((pad_e,), N, jnp.int32)])
    src3 = src_p.reshape(NTILES, CPT, CH)
    dst3 = dst_p.reshape(NTILES, CPT, CH)
    ea_p = jnp.concatenate(
        [edge_attr, jnp.zeros((pad_e, DE), jnp.float32)], axis=0)

    pad_n = NP_ - N
    x_ue_p = jnp.concatenate([x_ue, jnp.zeros((pad_n, D), jnp.float32)])
    x_ap_p = jnp.concatenate([x_ap, jnp.zeros((pad_n, D), jnp.float32)])

    W1a, W1b = W1[:DE], W1[DE:]
    W2a, W2b = W2[:D], W2[D:]

    H = _tc_matmul(x_ue_p, W1b, 512)
    P = _tc_matmul_bias(ea_p, W1a, b1.reshape(1, D), 4096)
    partials = _sc_aggregate(H, P, src3, dst3)
    out = _tc_final(x_ap_p, partials, W2a, W2b, b2.reshape(1, D), 512)
    return out[:N]


# speedup vs baseline: 2.1778x; 2.1778x over previous
import functools

import jax
import jax.numpy as jnp
from jax import lax
from jax.experimental import pallas as pl
from jax.experimental.pallas import tpu as pltpu
from jax.experimental.pallas import tpu_sc as plsc

N = 10000
E = 320000
D = 128
DE = 16

NP_ = 10240
CH = 128
NTILES = 32
CPT = 80
EPT = CPT * CH
EP = NTILES * EPT
GRP = 16
RPS = NP_ // 16

_mesh = plsc.VectorSubcoreMesh(
    core_axis_name="c", subcore_axis_name="s", num_cores=2, num_subcores=16
)


def _sc_body(h_hbm, p_hbm, src_hbm, dst_hbm, out_hbm,
             acc, src_v, dst_v, g_v, p_v, sem):
    c = lax.axis_index("c")
    s = lax.axis_index("s")
    wid = s * 2 + c

    zeros16 = jnp.zeros((16,), jnp.float32)

    @pl.loop(0, CH)
    def _(r):
        for cc in range(8):
            g_v[r, pl.ds(cc * 16, 16)] = zeros16

    for t in range(RPS // CH):
        pltpu.sync_copy(g_v, acc.at[pl.ds(s * RPS + t * CH, CH)])
    plsc.subcore_barrier()

    ebase = wid * EPT

    @pl.loop(0, CPT // GRP)
    def _(grp):
        pltpu.sync_copy(src_hbm.at[wid, pl.ds(grp * GRP, GRP)], src_v)
        pltpu.sync_copy(dst_hbm.at[wid, pl.ds(grp * GRP, GRP)], dst_v)

        @pl.loop(0, GRP)
        def _(j):
            cidx = grp * GRP + j
            pltpu.async_copy(h_hbm.at[src_v.at[j]], g_v, sem).wait()
            pltpu.sync_copy(p_hbm.at[pl.ds(ebase + cidx * CH, CH)], p_v)

            @pl.loop(0, CH)
            def _(r):
                for cc in range(8):
                    sl = pl.ds(cc * 16, 16)
                    g_v[r, sl] = jnp.maximum(g_v[r, sl] + p_v[r, sl], 0.0)

            pltpu.sync_copy(g_v, acc.at[dst_v.at[j]], add=True)

    plsc.subcore_barrier()
    pltpu.sync_copy(acc.at[pl.ds(s * RPS, RPS)],
                    out_hbm.at[pl.ds(c * NP_ + s * RPS, RPS)])


_sc_aggregate = functools.partial(
    pl.kernel,
    out_type=jax.ShapeDtypeStruct((2 * NP_, D), jnp.float32),
    mesh=_mesh,
    scratch_types=[
        pltpu.VMEM_SHARED((NP_, D), jnp.float32),
        pltpu.VMEM((GRP, CH), jnp.int32),
        pltpu.VMEM((GRP, CH), jnp.int32),
        pltpu.VMEM((CH, D), jnp.float32),
        pltpu.VMEM((CH, D), jnp.float32),
        pltpu.SemaphoreType.DMA,
    ],
)(_sc_body)


def _mm_body(x_ref, w_ref, o_ref):
    o_ref[...] = jax.lax.dot_general(
        x_ref[...], w_ref[...], (((1,), (0,)), ((), ())),
        preferred_element_type=jnp.float32)


def _mm_bias_body(x_ref, w_ref, b_ref, o_ref):
    o_ref[...] = jax.lax.dot_general(
        x_ref[...], w_ref[...], (((1,), (0,)), ((), ())),
        preferred_element_type=jnp.float32) + b_ref[...]


def _final_body(x_ref, a0_ref, a1_ref, wa_ref, wb_ref, b_ref, o_ref):
    agg = a0_ref[...] + a1_ref[...]
    acc = jax.lax.dot_general(
        x_ref[...], wa_ref[...], (((1,), (0,)), ((), ())),
        preferred_element_type=jnp.float32)
    acc += jax.lax.dot_general(
        agg, wb_ref[...], (((1,), (0,)), ((), ())),
        preferred_element_type=jnp.float32)
    o_ref[...] = jnp.maximum(acc + b_ref[...], 0.0)


def _tc_matmul(x, w, block_rows):
    m = x.shape[0]
    return pl.pallas_call(
        _mm_body,
        grid=(m // block_rows,),
        in_specs=[
            pl.BlockSpec((block_rows, x.shape[1]), lambda i: (i, 0)),
            pl.BlockSpec((w.shape[0], w.shape[1]), lambda i: (0, 0)),
        ],
        out_specs=pl.BlockSpec((block_rows, w.shape[1]), lambda i: (i, 0)),
        out_shape=jax.ShapeDtypeStruct((m, w.shape[1]), jnp.float32),
    )(x, w)


def _tc_matmul_bias(x, w, b, block_rows):
    m = x.shape[0]
    return pl.pallas_call(
        _mm_bias_body,
        grid=(m // block_rows,),
        in_specs=[
            pl.BlockSpec((block_rows, x.shape[1]), lambda i: (i, 0)),
            pl.BlockSpec((w.shape[0], w.shape[1]), lambda i: (0, 0)),
            pl.BlockSpec((1, w.shape[1]), lambda i: (0, 0)),
        ],
        out_specs=pl.BlockSpec((block_rows, w.shape[1]), lambda i: (i, 0)),
        out_shape=jax.ShapeDtypeStruct((m, w.shape[1]), jnp.float32),
    )(x, w, b)


def _tc_final(x_ap, partials, wa, wb, b, block_rows):
    nblk = NP_ // block_rows
    return pl.pallas_call(
        _final_body,
        grid=(nblk,),
        in_specs=[
            pl.BlockSpec((block_rows, D), lambda i: (i, 0)),
            pl.BlockSpec((block_rows, D), lambda i: (i, 0)),
            pl.BlockSpec((block_rows, D), lambda i: (i + NP_ // block_rows, 0)),
            pl.BlockSpec((D, D), lambda i: (0, 0)),
            pl.BlockSpec((D, D), lambda i: (0, 0)),
            pl.BlockSpec((1, D), lambda i: (0, 0)),
        ],
        out_specs=pl.BlockSpec((block_rows, D), lambda i: (i, 0)),
        out_shape=jax.ShapeDtypeStruct((NP_, D), jnp.float32),
    )(x_ap, partials, partials, wa, wb, b)


def kernel(x_ue, x_ap, edge_index, edge_attr, W1, b1, W2, b2):
    src = edge_index[0].astype(jnp.int32)
    dst = edge_index[1].astype(jnp.int32)

    pad_e = EP - E
    src_p = jnp.concatenate([src, jnp.zeros((pad_e,), jnp.int32)])
    dst_p = jnp.concatenate([dst, jnp.full((pad_e,), N, jnp.int32)])
    src3 = src_p.reshape(NTILES, CPT, CH)
    dst3 = dst_p.reshape(NTILES, CPT, CH)
    ea_p = jnp.concatenate(
        [edge_attr, jnp.zeros((pad_e, DE), jnp.float32)], axis=0)

    pad_n = NP_ - N
    x_ue_p = jnp.concatenate([x_ue, jnp.zeros((pad_n, D), jnp.float32)])
    x_ap_p = jnp.concatenate([x_ap, jnp.zeros((pad_n, D), jnp.float32)])

    W1a, W1b = W1[:DE], W1[DE:]
    W2a, W2b = W2[:D], W2[D:]

    H = _tc_matmul(x_ue_p, W1b, 512)
    P = _tc_matmul_bias(ea_p, W1a, b1.reshape(1, D), 4096)
    partials = _sc_aggregate(H, P, src3, dst3)
    out = _tc_final(x_ap_p, partials, W2a, W2b, b2.reshape(1, D), 512)
    return out[:N]
